# baseline (speedup 1.0000x reference)
import functools

import jax
import jax.numpy as jnp
from jax import lax
from jax.experimental import pallas as pl
from jax.experimental.pallas import tpu as pltpu
from jax.experimental.pallas import tpu_sc as plsc

N = 10000
E = 160000
D = 256
R = 200

NPAD = 10240
NB = 1024
RL = 128
RH = R - RL
LSZ = NPAD * RL
HSZ = RH * NPAD
NR = LSZ + HSZ
N_TILES = 16
ROWS = 80
CHUNK = 128
EPT = ROWS * CHUNK
EPAD = N_TILES * EPT
ZCHUNK = NR // N_TILES


GROUP = 4
N_GROUPS = ROWS // GROUP
DUMMY = NR


def _count_body(pk_hbm, zeros_hbm, ones_hbm, out_hbm,
                pk_a, pk_b, key_a, key_b, one_v, cnt_sh,
                sem_a, sem_b, sem_s):
    c = lax.axis_index("c")
    s = lax.axis_index("s")

    with jax.named_scope("cnt_init"):
        pltpu.sync_copy(ones_hbm, one_v)
        pltpu.async_copy(pk_hbm.at[s, pl.ds(0, GROUP)], pk_a, sem_a)
        pltpu.sync_copy(zeros_hbm, cnt_sh.at[pl.ds(s * ZCHUNK, ZCHUNK)])
        plsc.subcore_barrier()

    tgt_v = jnp.full((16,), 2, jnp.int32) - lax.broadcast(c, (16,))
    two_v = jnp.full((16,), 2, jnp.int32)
    three_v = jnp.full((16,), 3, jnp.int32)
    dumbase_v = jnp.full((16,), DUMMY, jnp.int32)
    m1023_v = jnp.full((16,), 1023, jnp.int32)

    def keys_from(pk_v, key_v):
        for jr in range(GROUP):
            for jc in range(CHUNK // 16):
                w = pk_v[jr, pl.ds(jc * 16, 16)]
                k = lax.shift_right_logical(w, two_v)
                mb = lax.bitwise_and(w, three_v)
                d = dumbase_v + lax.bitwise_and(k, m1023_v)
                key_v[pl.ds(jr * CHUNK + jc * 16, 16)] = jnp.where(
                    mb == tgt_v, k, d)

    def pair(j, carry):
        g0 = 2 * j
        pltpu.make_async_copy(pk_hbm.at[s, pl.ds(g0 * GROUP, GROUP)],
                              pk_a, sem_a).wait()
        pltpu.async_copy(pk_hbm.at[s, pl.ds((g0 + 1) * GROUP, GROUP)],
                         pk_b, sem_b)
        keys_from(pk_a, key_a)
        descs = [pltpu.async_copy(one_v, cnt_sh.at[key_a], sem_s, add=True)]

        pltpu.make_async_copy(pk_hbm.at[s, pl.ds((g0 + 1) * GROUP, GROUP)],
                              pk_b, sem_b).wait()

        @pl.when(g0 + 2 < N_GROUPS)
        def _():
            pltpu.async_copy(pk_hbm.at[s, pl.ds((g0 + 2) * GROUP, GROUP)],
                             pk_a, sem_a)

        keys_from(pk_b, key_b)
        descs += [pltpu.async_copy(one_v, cnt_sh.at[key_b], sem_s, add=True)]
        for d in descs:
            d.wait()
        return carry

    with jax.named_scope("cnt_scan"):
        lax.fori_loop(0, N_GROUPS // 2, pair, 0)

        plsc.subcore_barrier()

    with jax.named_scope("cnt_out"):
        pltpu.sync_copy(cnt_sh.at[pl.ds(s * ZCHUNK, ZCHUNK)],
                        out_hbm.at[pl.ds(c * NR + s * ZCHUNK, ZCHUNK)])


_count_kernel = functools.partial(
    pl.kernel,
    out_type=jax.ShapeDtypeStruct((2 * NR,), jnp.float32),
    mesh=plsc.VectorSubcoreMesh(core_axis_name="c", subcore_axis_name="s"),
    scratch_types=[
        pltpu.VMEM((GROUP, CHUNK), jnp.int32),
        pltpu.VMEM((GROUP, CHUNK), jnp.int32),
        pltpu.VMEM((GROUP * CHUNK,), jnp.int32),
        pltpu.VMEM((GROUP * CHUNK,), jnp.int32),
        pltpu.VMEM((GROUP * CHUNK,), jnp.float32),
        pltpu.VMEM_SHARED((NR + 1040,), jnp.float32),
        pltpu.SemaphoreType.DMA,
        pltpu.SemaphoreType.DMA,
        pltpu.SemaphoreType.DMA,
    ],
)(_count_body)


_HI = jax.lax.Precision.HIGHEST


def _s_body(ent_ref, rell_ref, relh_ref, sl_ref, sh_ref):
    ent = ent_ref[...]
    sl_ref[...] = lax.dot_general(ent, rell_ref[...],
                                  (((1,), (1,)), ((), ())), precision=_HI)
    sh_ref[...] = lax.dot_general(relh_ref[...], ent,
                                  (((1,), (1,)), ((), ())), precision=_HI)


def _tr(v):
    return lax.transpose(v, (1, 0))


def _main_body(sl_ref, sh_ref, cl_ref, ch_ref, rel_ref, wi_ref, wo_ref,
               bi_ref, bo_ref, neigh_ref, stats_ref, hi_s, ho_s):
    i = pl.program_id(0)

    @pl.when(i == 0)
    def _init():
        rel = rel_ref[...]
        hi_s[...] = lax.dot_general(rel, wi_ref[...], (((1,), (1,)), ((), ())),
                                    precision=_HI) + bi_ref[...]
        ho_s[...] = lax.dot_general(rel, wo_ref[...], (((1,), (1,)), ((), ())),
                                    precision=_HI) + bo_ref[...]
        stats_ref[...] = jnp.zeros((8, D), jnp.float32)

    neg = jnp.float32(-1e30)
    sl = sl_ref[...]
    sh = sh_ref[...]
    cil, col = cl_ref[0], cl_ref[1]
    cih, coh = ch_ref[0], ch_ref[1]
    cl = cil + col
    ch = cih + coh
    tl = jnp.where(cl > 0.0, sl, neg)
    th = jnp.where(ch > 0.0, sh, neg)
    mx = jnp.maximum(jnp.max(tl, axis=1, keepdims=True),
                     _tr(jnp.max(th, axis=0, keepdims=True)))
    mx_c = _tr(mx)
    exl = jnp.exp(tl - mx)
    exh = jnp.exp(th - mx_c)
    wl = cl * exl
    wh = ch * exh
    denom = (jnp.sum(wl, axis=1, keepdims=True)
             + _tr(jnp.sum(wh, axis=0, keepdims=True)))
    dsafe = jnp.where(denom > 0.0, denom, 1.0)
    dsafe_c = _tr(dsafe)
    pil = cil * exl / dsafe
    pol = col * exl / dsafe
    pih = cih * exh / dsafe_c
    poh = coh * exh / dsafe_c
    neigh = (lax.dot_general(pil, hi_s[0:RL], (((1,), (0,)), ((), ())))
             + lax.dot_general(pol, ho_s[0:RL], (((1,), (0,)), ((), ())))
             + lax.dot_general(pih, hi_s[RL:R], (((0,), (0,)), ((), ())))
             + lax.dot_general(poh, ho_s[RL:R], (((0,), (0,)), ((), ()))))
    neigh_ref[...] = neigh
    stats_ref[0:1, :] = stats_ref[0:1, :] + jnp.sum(neigh, axis=0,
                                                    keepdims=True)
    stats_ref[1:2, :] = stats_ref[1:2, :] + jnp.sum(neigh * neigh, axis=0,
                                                    keepdims=True)


def _bn_body(neigh_ref, stats_ref, gamma_ref, beta_ref, out_ref):
    mean = stats_ref[0:1, :] / jnp.float32(N)
    var = stats_ref[1:2, :] / jnp.float32(N) - mean * mean
    inv = lax.rsqrt(var + 1e-5)
    out_ref[...] = jnp.tanh((neigh_ref[...] - mean) * inv * gamma_ref[...]
                            + beta_ref[...])


def kernel(ent_emb, rel_emb, W_o, b_o, W_i, b_i, gamma, beta, edge_index,
           etype, in_edges_mask):
    dst = edge_index[1].astype(jnp.int32)
    ety = etype.astype(jnp.int32)
    msk = in_edges_mask.astype(jnp.int32)

    pad = EPAD - E
    key = jnp.where(ety < RL, dst * RL + ety,
                    LSZ + (ety - RL) * NPAD + dst)
    packed = key * 4 + jnp.where(msk > 0, 2, 1)
    pk3 = jnp.pad(packed, (0, pad)).reshape(N_TILES, ROWS, CHUNK)
    zeros = jnp.zeros((ZCHUNK,), jnp.float32)
    ones = jnp.ones((GROUP * CHUNK,), jnp.float32)

    grid = NPAD // NB
    ent_pad = jnp.pad(ent_emb, ((0, NPAD - N), (0, 0)))
    SL, SH = pl.pallas_call(
        _s_body,
        grid=(grid,),
        in_specs=[
            pl.BlockSpec((NB, D), lambda i: (i, 0)),
            pl.BlockSpec((RL, D), lambda i: (0, 0)),
            pl.BlockSpec((RH, D), lambda i: (0, 0)),
        ],
        out_specs=[
            pl.BlockSpec((NB, RL), lambda i: (i, 0)),
            pl.BlockSpec((RH, NB), lambda i: (0, i)),
        ],
        out_shape=[
            jax.ShapeDtypeStruct((NPAD, RL), jnp.float32),
            jax.ShapeDtypeStruct((RH, NPAD), jnp.float32),
        ],
    )(ent_pad, rel_emb[:RL], rel_emb[RL:])

    counts = _count_kernel(pk3, zeros, ones).reshape(2, NR)
    cnt_l = counts[:, :LSZ].reshape(2, NPAD, RL)
    cnt_h = counts[:, LSZ:].reshape(2, RH, NPAD)

    neigh, stats = pl.pallas_call(
        _main_body,
        grid=(grid,),
        in_specs=[
            pl.BlockSpec((NB, RL), lambda i: (i, 0)),
            pl.BlockSpec((RH, NB), lambda i: (0, i)),
            pl.BlockSpec((2, NB, RL), lambda i: (0, i, 0)),
            pl.BlockSpec((2, RH, NB), lambda i: (0, 0, i)),
            pl.BlockSpec((R, D), lambda i: (0, 0)),
            pl.BlockSpec((D, D), lambda i: (0, 0)),
            pl.BlockSpec((D, D), lambda i: (0, 0)),
            pl.BlockSpec((1, D), lambda i: (0, 0)),
            pl.BlockSpec((1, D), lambda i: (0, 0)),
        ],
        out_specs=[
            pl.BlockSpec((NB, D), lambda i: (i, 0)),
            pl.BlockSpec((8, D), lambda i: (0, 0)),
        ],
        out_shape=[
            jax.ShapeDtypeStruct((NPAD, D), jnp.float32),
            jax.ShapeDtypeStruct((8, D), jnp.float32),
        ],
        scratch_shapes=[
            pltpu.VMEM((R, D), jnp.float32),
            pltpu.VMEM((R, D), jnp.float32),
        ],
    )(SL, SH, cnt_l, cnt_h, rel_emb, W_i, W_o,
      b_i.reshape(1, D), b_o.reshape(1, D))

    out = pl.pallas_call(
        _bn_body,
        grid=(grid,),
        in_specs=[
            pl.BlockSpec((1000, D), lambda i: (i, 0)),
            pl.BlockSpec((8, D), lambda i: (0, 0)),
            pl.BlockSpec((1, D), lambda i: (0, 0)),
            pl.BlockSpec((1, D), lambda i: (0, 0)),
        ],
        out_specs=pl.BlockSpec((1000, D), lambda i: (i, 0)),
        out_shape=jax.ShapeDtypeStruct((N, D), jnp.float32),
    )(neigh, stats, gamma.reshape(1, D), beta.reshape(1, D))

    return out


# speedup vs baseline: 1.6324x; 1.6324x over previous
import functools

import jax
import jax.numpy as jnp
from jax import lax
from jax.experimental import pallas as pl
from jax.experimental.pallas import tpu as pltpu
from jax.experimental.pallas import tpu_sc as plsc

N = 10000
E = 160000
D = 256
R = 200

NPAD = 10240
NB = 1024
RL = 128
RH = R - RL
LSZ = NPAD * RL
HSZ = RH * NPAD
NR = LSZ + HSZ
N_TILES = 16
ROWS = 80
CHUNK = 128
EPT = ROWS * CHUNK
EPAD = N_TILES * EPT
ZCHUNK = NR // N_TILES


GROUP = 4
N_GROUPS = ROWS // GROUP
DUMMY = NR


def _count_body(pk_hbm, zeros_hbm, ones_hbm, outl_i, outh_i, outl_o, outh_o,
                pk_a, pk_b, key_a, key_b, one_v, cnt_sh,
                sem_a, sem_b, sem_s):
    c = lax.axis_index("c")
    s = lax.axis_index("s")

    with jax.named_scope("cnt_init"):
        pltpu.sync_copy(ones_hbm, one_v)
        pltpu.async_copy(pk_hbm.at[s, pl.ds(0, GROUP)], pk_a, sem_a)
        pltpu.sync_copy(zeros_hbm, cnt_sh.at[pl.ds(s * ZCHUNK, ZCHUNK)])
        plsc.subcore_barrier()

    tgt_v = jnp.full((16,), 2, jnp.int32) - lax.broadcast(c, (16,))
    two_v = jnp.full((16,), 2, jnp.int32)
    three_v = jnp.full((16,), 3, jnp.int32)
    dumbase_v = jnp.full((16,), DUMMY, jnp.int32)
    m1023_v = jnp.full((16,), 1023, jnp.int32)

    def keys_from(pk_v, key_v):
        for jr in range(GROUP):
            for jc in range(CHUNK // 16):
                w = pk_v[jr, pl.ds(jc * 16, 16)]
                k = lax.shift_right_logical(w, two_v)
                mb = lax.bitwise_and(w, three_v)
                d = dumbase_v + lax.bitwise_and(k, m1023_v)
                key_v[pl.ds(jr * CHUNK + jc * 16, 16)] = jnp.where(
                    mb == tgt_v, k, d)

    def pair(j, carry):
        g0 = 2 * j
        pltpu.make_async_copy(pk_hbm.at[s, pl.ds(g0 * GROUP, GROUP)],
                              pk_a, sem_a).wait()
        pltpu.async_copy(pk_hbm.at[s, pl.ds((g0 + 1) * GROUP, GROUP)],
                         pk_b, sem_b)
        keys_from(pk_a, key_a)
        descs = [pltpu.async_copy(one_v, cnt_sh.at[key_a], sem_s, add=True)]

        pltpu.make_async_copy(pk_hbm.at[s, pl.ds((g0 + 1) * GROUP, GROUP)],
                              pk_b, sem_b).wait()

        @pl.when(g0 + 2 < N_GROUPS)
        def _():
            pltpu.async_copy(pk_hbm.at[s, pl.ds((g0 + 2) * GROUP, GROUP)],
                             pk_a, sem_a)

        keys_from(pk_b, key_b)
        descs += [pltpu.async_copy(one_v, cnt_sh.at[key_b], sem_s, add=True)]
        for d in descs:
            d.wait()
        return carry

    with jax.named_scope("cnt_scan"):
        lax.fori_loop(0, N_GROUPS // 2, pair, 0)

        plsc.subcore_barrier()

    LREM = LSZ - 10 * ZCHUNK
    HREM = ZCHUNK - LREM
    with jax.named_scope("cnt_out"):
        @pl.when(c == 0)
        def _out_i():
            _readback(s, cnt_sh, outl_i, outh_i, LREM, HREM)

        @pl.when(c == 1)
        def _out_o():
            _readback(s, cnt_sh, outl_o, outh_o, LREM, HREM)


def _readback(s, cnt_sh, outl, outh, lrem, hrem):
    @pl.when(s < 10)
    def _l():
        pltpu.sync_copy(cnt_sh.at[pl.ds(s * ZCHUNK, ZCHUNK)],
                        outl.at[pl.ds(s * ZCHUNK, ZCHUNK)])

    @pl.when(s == 10)
    def _split():
        pltpu.sync_copy(cnt_sh.at[pl.ds(10 * ZCHUNK, lrem)],
                        outl.at[pl.ds(10 * ZCHUNK, lrem)])
        pltpu.sync_copy(cnt_sh.at[pl.ds(LSZ, hrem)],
                        outh.at[pl.ds(0, hrem)])

    @pl.when(s > 10)
    def _h():
        off = s * ZCHUNK - LSZ
        pltpu.sync_copy(cnt_sh.at[pl.ds(s * ZCHUNK, ZCHUNK)],
                        outh.at[pl.ds(off, ZCHUNK)])


_count_kernel = functools.partial(
    pl.kernel,
    out_type=[
        jax.ShapeDtypeStruct((LSZ,), jnp.float32),
        jax.ShapeDtypeStruct((HSZ,), jnp.float32),
        jax.ShapeDtypeStruct((LSZ,), jnp.float32),
        jax.ShapeDtypeStruct((HSZ,), jnp.float32),
    ],
    mesh=plsc.VectorSubcoreMesh(core_axis_name="c", subcore_axis_name="s"),
    scratch_types=[
        pltpu.VMEM((GROUP, CHUNK), jnp.int32),
        pltpu.VMEM((GROUP, CHUNK), jnp.int32),
        pltpu.VMEM((GROUP * CHUNK,), jnp.int32),
        pltpu.VMEM((GROUP * CHUNK,), jnp.int32),
        pltpu.VMEM((GROUP * CHUNK,), jnp.float32),
        pltpu.VMEM_SHARED((NR + 1040,), jnp.float32),
        pltpu.SemaphoreType.DMA,
        pltpu.SemaphoreType.DMA,
        pltpu.SemaphoreType.DMA,
    ],
)(_count_body)


_HI = jax.lax.Precision.HIGHEST


def _s_body(ent_ref, rell_ref, relh_ref, sl_ref, sh_ref):
    ent = ent_ref[...]
    sl_ref[...] = lax.dot_general(ent, rell_ref[...],
                                  (((1,), (1,)), ((), ())), precision=_HI)
    sh_ref[...] = lax.dot_general(relh_ref[...], ent,
                                  (((1,), (1,)), ((), ())), precision=_HI)


def _tr(v):
    return lax.transpose(v, (1, 0))


def _main_body(sl_ref, sh_ref, cil_ref, cih_ref, col_ref, coh_ref,
               rel_ref, wi_ref, wo_ref, bi_ref, bo_ref,
               neigh_ref, stats_ref, hi_s, ho_s):
    i = pl.program_id(0)

    @pl.when(i == 0)
    def _init():
        rel = rel_ref[...]
        hi_s[...] = lax.dot_general(rel, wi_ref[...], (((1,), (1,)), ((), ())),
                                    precision=_HI) + bi_ref[...]
        ho_s[...] = lax.dot_general(rel, wo_ref[...], (((1,), (1,)), ((), ())),
                                    precision=_HI) + bo_ref[...]
        stats_ref[...] = jnp.zeros((8, D), jnp.float32)

    neg = jnp.float32(-1e30)
    sl = sl_ref[...]
    sh = sh_ref[...]
    cil, col = cil_ref[...], col_ref[...]
    cih, coh = cih_ref[...], coh_ref[...]
    cl = cil + col
    ch = cih + coh
    tl = jnp.where(cl > 0.0, sl, neg)
    th = jnp.where(ch > 0.0, sh, neg)
    mx = jnp.maximum(jnp.max(tl, axis=1, keepdims=True),
                     _tr(jnp.max(th, axis=0, keepdims=True)))
    mx_c = _tr(mx)
    exl = jnp.exp(tl - mx)
    exh = jnp.exp(th - mx_c)
    wl = cl * exl
    wh = ch * exh
    denom = (jnp.sum(wl, axis=1, keepdims=True)
             + _tr(jnp.sum(wh, axis=0, keepdims=True)))
    dsafe = jnp.where(denom > 0.0, denom, 1.0)
    dsafe_c = _tr(dsafe)
    pil = cil * exl / dsafe
    pol = col * exl / dsafe
    pih = cih * exh / dsafe_c
    poh = coh * exh / dsafe_c
    neigh = (lax.dot_general(pil, hi_s[0:RL], (((1,), (0,)), ((), ())))
             + lax.dot_general(pol, ho_s[0:RL], (((1,), (0,)), ((), ())))
             + lax.dot_general(pih, hi_s[RL:R], (((0,), (0,)), ((), ())))
             + lax.dot_general(poh, ho_s[RL:R], (((0,), (0,)), ((), ()))))
    neigh_ref[...] = neigh
    stats_ref[0:1, :] = stats_ref[0:1, :] + jnp.sum(neigh, axis=0,
                                                    keepdims=True)
    stats_ref[1:2, :] = stats_ref[1:2, :] + jnp.sum(neigh * neigh, axis=0,
                                                    keepdims=True)


def _bn_body(neigh_ref, stats_ref, gamma_ref, beta_ref, out_ref):
    mean = stats_ref[0:1, :] / jnp.float32(N)
    var = stats_ref[1:2, :] / jnp.float32(N) - mean * mean
    inv = lax.rsqrt(var + 1e-5)
    out_ref[...] = jnp.tanh((neigh_ref[...] - mean) * inv * gamma_ref[...]
                            + beta_ref[...])


def kernel(ent_emb, rel_emb, W_o, b_o, W_i, b_i, gamma, beta, edge_index,
           etype, in_edges_mask):
    dst = edge_index[1].astype(jnp.int32)
    ety = etype.astype(jnp.int32)
    msk = in_edges_mask.astype(jnp.int32)

    pad = EPAD - E
    key = jnp.where(ety < RL, dst * RL + ety,
                    LSZ + (ety - RL) * NPAD + dst)
    packed = key * 4 + jnp.where(msk > 0, 2, 1)
    pk3 = jnp.pad(packed, (0, pad)).reshape(N_TILES, ROWS, CHUNK)
    zeros = jnp.zeros((ZCHUNK,), jnp.float32)
    ones = jnp.ones((GROUP * CHUNK,), jnp.float32)

    grid = NPAD // NB
    SL, SH = pl.pallas_call(
        _s_body,
        grid=(grid,),
        in_specs=[
            pl.BlockSpec((NB, D), lambda i: (i, 0)),
            pl.BlockSpec((RL, D), lambda i: (0, 0)),
            pl.BlockSpec((RH, D), lambda i: (0, 0)),
        ],
        out_specs=[
            pl.BlockSpec((NB, RL), lambda i: (i, 0)),
            pl.BlockSpec((RH, NB), lambda i: (0, i)),
        ],
        out_shape=[
            jax.ShapeDtypeStruct((NPAD, RL), jnp.float32),
            jax.ShapeDtypeStruct((RH, NPAD), jnp.float32),
        ],
    )(ent_emb, rel_emb[:RL], rel_emb[RL:])

    cil, cih, col, coh = _count_kernel(pk3, zeros, ones)
    cil = cil.reshape(NPAD, RL)
    col = col.reshape(NPAD, RL)
    cih = cih.reshape(RH, NPAD)
    coh = coh.reshape(RH, NPAD)

    neigh, stats = pl.pallas_call(
        _main_body,
        grid=(grid,),
        in_specs=[
            pl.BlockSpec((NB, RL), lambda i: (i, 0)),
            pl.BlockSpec((RH, NB), lambda i: (0, i)),
            pl.BlockSpec((NB, RL), lambda i: (i, 0)),
            pl.BlockSpec((RH, NB), lambda i: (0, i)),
            pl.BlockSpec((NB, RL), lambda i: (i, 0)),
            pl.BlockSpec((RH, NB), lambda i: (0, i)),
            pl.BlockSpec((R, D), lambda i: (0, 0)),
            pl.BlockSpec((D, D), lambda i: (0, 0)),
            pl.BlockSpec((D, D), lambda i: (0, 0)),
            pl.BlockSpec((1, D), lambda i: (0, 0)),
            pl.BlockSpec((1, D), lambda i: (0, 0)),
        ],
        out_specs=[
            pl.BlockSpec((NB, D), lambda i: (i, 0)),
            pl.BlockSpec((8, D), lambda i: (0, 0)),
        ],
        out_shape=[
            jax.ShapeDtypeStruct((NPAD, D), jnp.float32),
            jax.ShapeDtypeStruct((8, D), jnp.float32),
        ],
        scratch_shapes=[
            pltpu.VMEM((R, D), jnp.float32),
            pltpu.VMEM((R, D), jnp.float32),
        ],
    )(SL, SH, cil, cih, col, coh, rel_emb, W_i, W_o,
      b_i.reshape(1, D), b_o.reshape(1, D))

    out = pl.pallas_call(
        _bn_body,
        grid=(grid,),
        in_specs=[
            pl.BlockSpec((1000, D), lambda i: (i, 0)),
            pl.BlockSpec((8, D), lambda i: (0, 0)),
            pl.BlockSpec((1, D), lambda i: (0, 0)),
            pl.BlockSpec((1, D), lambda i: (0, 0)),
        ],
        out_specs=pl.BlockSpec((1000, D), lambda i: (i, 0)),
        out_shape=jax.ShapeDtypeStruct((N, D), jnp.float32),
    )(neigh, stats, gamma.reshape(1, D), beta.reshape(1, D))

    return out
